# Initial kernel scaffold; baseline (speedup 1.0000x reference)
import jax
import jax.numpy as jnp
from jax.experimental import pallas as pl


def kernel(nodes, uv_neigh, graph_neigh, v_table, u_table, W, b):
    raise NotImplementedError("write your pallas kernel here")


# speedup vs baseline: 1.7291x; 1.7291x over previous
import functools

import jax
import jax.numpy as jnp
from jax import lax
from jax.experimental import pallas as pl
from jax.experimental.pallas import tpu as pltpu
from jax.experimental.pallas import tpu_sc as plsc

_B = 10000
_K = 64
_D = 128
_L = 16
_NC = 2
_NS = 16
_NW = _NC * _NS
_BP = 10240
_BW = _BP // _NW
_SELF_CHUNK = 80
_RCHUNK = 16
_GCHUNK = 2


def _sc_body(nodes_hbm, uv_hbm, gr_hbm, vtab_hbm, utab_hbm,
             self_out, uv_out, gr_out,
             idx_v, rows_v, acc_v, sidx_v, srows_v, sem):
    wid = lax.axis_index("s") * _NC + lax.axis_index("c")
    base = wid * _BW

    @pl.loop(0, _BW, step=_SELF_CHUNK)
    def _(r0):
        pltpu.sync_copy(nodes_hbm.at[pl.ds(base + r0, _SELF_CHUNK)], sidx_v)
        pltpu.async_copy(vtab_hbm.at[sidx_v], srows_v, sem).wait()
        pltpu.sync_copy(srows_v, self_out.at[pl.ds(base + r0, _SELF_CHUNK)])

    def seg_sum(idx_hbm, tab_hbm, out_hbm):
        @pl.loop(0, _BW, step=_RCHUNK)
        def _(r0):
            pltpu.sync_copy(
                idx_hbm.at[pl.ds((base + r0) * _K, _RCHUNK * _K)], idx_v)

            @pl.loop(0, _RCHUNK, step=_GCHUNK)
            def _(g0):
                pltpu.async_copy(
                    tab_hbm.at[idx_v.at[pl.ds(g0 * _K, _GCHUNK * _K)]],
                    rows_v, sem).wait()
                for rr in range(_GCHUNK):
                    for c in range(_D // _L):
                        def body(j, acc, rr=rr, c=c):
                            return acc + rows_v[rr * _K + j, pl.ds(c * _L, _L)]
                        acc = lax.fori_loop(
                            0, _K, body, jnp.zeros((_L,), jnp.float32))
                        acc_v[g0 + rr, pl.ds(c * _L, _L)] = acc

            pltpu.sync_copy(acc_v, out_hbm.at[pl.ds(base + r0, _RCHUNK)])

    seg_sum(uv_hbm, utab_hbm, uv_out)
    seg_sum(gr_hbm, vtab_hbm, gr_out)


def _tc_body(self_ref, us_ref, gs_ref, w1_ref, w2_ref, w3_ref, b_ref, o_ref):
    s = self_ref[...]
    u = us_ref[...]
    g = gs_ref[...] * (1.0 / _K)
    g = jnp.where(g >= 0.0, jnp.floor(g), jnp.ceil(g))
    acc = jnp.dot(s, w1_ref[...], preferred_element_type=jnp.float32)
    acc = acc + jnp.dot(u, w2_ref[...], preferred_element_type=jnp.float32)
    acc = acc + jnp.dot(g, w3_ref[...], preferred_element_type=jnp.float32)
    o_ref[...] = jnp.maximum(acc + b_ref[...], 0.0)


@jax.jit
def _run(nodes_p, uv_p, gr_p, v_table, u_table, w1, w2, w3, b2):
    mesh = plsc.VectorSubcoreMesh(core_axis_name="c", subcore_axis_name="s")
    f32 = jnp.float32
    sc = pl.kernel(
        _sc_body,
        out_type=[
            jax.ShapeDtypeStruct((_BP, _D), f32),
            jax.ShapeDtypeStruct((_BP, _D), f32),
            jax.ShapeDtypeStruct((_BP, _D), f32),
        ],
        mesh=mesh,
        scratch_types=[
            pltpu.VMEM((_RCHUNK * _K,), jnp.int32),
            pltpu.VMEM((_GCHUNK * _K, _D), f32),
            pltpu.VMEM((_RCHUNK, _D), f32),
            pltpu.VMEM((_SELF_CHUNK,), jnp.int32),
            pltpu.VMEM((_SELF_CHUNK, _D), f32),
            pltpu.SemaphoreType.DMA,
        ],
    )
    self_rows, uv_sum, gr_sum = sc(nodes_p, uv_p, gr_p, v_table, u_table)

    nblk = 10
    rows = _BP // nblk
    out = pl.pallas_call(
        _tc_body,
        grid=(nblk,),
        in_specs=[
            pl.BlockSpec((rows, _D), lambda i: (i, 0)),
            pl.BlockSpec((rows, _D), lambda i: (i, 0)),
            pl.BlockSpec((rows, _D), lambda i: (i, 0)),
            pl.BlockSpec((_D, _D), lambda i: (0, 0)),
            pl.BlockSpec((_D, _D), lambda i: (0, 0)),
            pl.BlockSpec((_D, _D), lambda i: (0, 0)),
            pl.BlockSpec((1, _D), lambda i: (0, 0)),
        ],
        out_specs=pl.BlockSpec((rows, _D), lambda i: (i, 0)),
        out_shape=jax.ShapeDtypeStruct((_BP, _D), f32),
    )(self_rows, uv_sum, gr_sum, w1, w2, w3, b2)
    return out[:_B]


def kernel(nodes, uv_neigh, graph_neigh, v_table, u_table, W, b):
    pad = _BP - _B
    nodes_p = jnp.pad(nodes, (0, pad))
    uv_p = jnp.pad(uv_neigh, ((0, pad), (0, 0))).reshape(_BP * _K)
    gr_p = jnp.pad(graph_neigh, ((0, pad), (0, 0))).reshape(_BP * _K)
    w1 = 0.3 * W[:_D]
    w2 = (0.4 / _K) * W[_D:2 * _D]
    w3 = 0.3 * W[2 * _D:]
    b2 = b.reshape(1, _D)
    return _run(nodes_p, uv_p, gr_p, v_table, u_table, w1, w2, w3, b2)


# speedup vs baseline: 3.1380x; 1.8148x over previous
import jax
import jax.numpy as jnp
from jax import lax
from jax.experimental import pallas as pl
from jax.experimental.pallas import tpu as pltpu
from jax.experimental.pallas import tpu_sc as plsc

_B = 10000
_K = 64
_D = 128
_L = 16
_NC = 2
_NS = 16
_NW = _NC * _NS
_BP = 10240
_BW = _BP // _NW
_GROWS = 2
_GIDX = _GROWS * _K
_NG = _BW // _GROWS


def _sc_body(nodes_hbm, uv_hbm, gr_hbm, vtab_hbm, utab_hbm,
             self_out, uv_out, gr_out,
             idx_v, buf0, buf1, stage_v, sem0, sem1):
    wid = lax.axis_index("s") * _NC + lax.axis_index("c")
    base = wid * _BW
    bufs = (buf0, buf1)
    sems = (sem0, sem1)

    pltpu.sync_copy(nodes_hbm.at[pl.ds(base, _BW)], idx_v.at[pl.ds(0, _BW)])
    c0 = pltpu.async_copy(
        vtab_hbm.at[idx_v.at[pl.ds(0, 128)]], stage_v.at[pl.ds(0, 128)], sem0)
    c1 = pltpu.async_copy(
        vtab_hbm.at[idx_v.at[pl.ds(128, 128)]], stage_v.at[pl.ds(128, 128)],
        sem1)
    c2 = pltpu.async_copy(
        vtab_hbm.at[idx_v.at[pl.ds(256, 64)]], stage_v.at[pl.ds(256, 64)],
        sem0)
    c0.wait()
    c2.wait()
    c1.wait()
    pltpu.sync_copy(stage_v, self_out.at[pl.ds(base, _BW)])

    def seg_sum(idx_hbm, tab_hbm, out_hbm):
        pltpu.sync_copy(idx_hbm.at[pl.ds(base * _K, _BW * _K)], idx_v)

        def fire(g, p):
            pltpu.async_copy(
                tab_hbm.at[idx_v.at[pl.ds(g * _GIDX, _GIDX)]], bufs[p],
                sems[p])

        def drain(g, p):
            pltpu.make_async_copy(
                tab_hbm.at[idx_v.at[pl.ds(g * _GIDX, _GIDX)]], bufs[p],
                sems[p]).wait()

        def accum_store(g, p):
            buf = bufs[p]
            for rr in range(_GROWS):
                def body(j, accs, rr=rr):
                    r0 = rr * _K + j * 8
                    accs = list(accs)
                    for u in range(8):
                        for c in range(_D // _L):
                            accs[c] = accs[c] + buf[r0 + u, pl.ds(c * _L, _L)]
                    return tuple(accs)
                accs = lax.fori_loop(
                    0, _K // 8, body,
                    tuple(jnp.zeros((_L,), jnp.float32)
                          for _ in range(_D // _L)))
                for c in range(_D // _L):
                    stage_v[g * _GROWS + rr, pl.ds(c * _L, _L)] = accs[c]

        fire(0, 0)
        fire(1, 1)

        @pl.loop(0, _NG - 2, step=2)
        def _(g):
            for p in range(2):
                gg = g + p
                drain(gg, p)
                accum_store(gg, p)
                fire(gg + 2, p)

        for p in range(2):
            gg = _NG - 2 + p
            drain(gg, p)
            accum_store(gg, p)

        pltpu.sync_copy(stage_v, out_hbm.at[pl.ds(base, _BW)])

    seg_sum(uv_hbm, utab_hbm, uv_out)
    seg_sum(gr_hbm, vtab_hbm, gr_out)


def _tc_body(self_ref, us_ref, gs_ref, w1_ref, w2_ref, w3_ref, b_ref, o_ref):
    s = self_ref[...]
    u = us_ref[...]
    g = gs_ref[...] * (1.0 / _K)
    g = jnp.where(g >= 0.0, jnp.floor(g), jnp.ceil(g))
    acc = jnp.dot(s, w1_ref[...], preferred_element_type=jnp.float32)
    acc = acc + jnp.dot(u, w2_ref[...], preferred_element_type=jnp.float32)
    acc = acc + jnp.dot(g, w3_ref[...], preferred_element_type=jnp.float32)
    o_ref[...] = jnp.maximum(acc + b_ref[...], 0.0)


@jax.jit
def _run(nodes_p, uv_p, gr_p, v_table, u_table, w1, w2, w3, b2):
    mesh = plsc.VectorSubcoreMesh(core_axis_name="c", subcore_axis_name="s")
    f32 = jnp.float32
    sc = pl.kernel(
        _sc_body,
        out_type=[
            jax.ShapeDtypeStruct((_BP, _D), f32),
            jax.ShapeDtypeStruct((_BP, _D), f32),
            jax.ShapeDtypeStruct((_BP, _D), f32),
        ],
        mesh=mesh,
        scratch_types=[
            pltpu.VMEM((_BW * _K,), jnp.int32),
            pltpu.VMEM((_GIDX, _D), f32),
            pltpu.VMEM((_GIDX, _D), f32),
            pltpu.VMEM((_BW, _D), f32),
            pltpu.SemaphoreType.DMA,
            pltpu.SemaphoreType.DMA,
        ],
    )
    self_rows, uv_sum, gr_sum = sc(nodes_p, uv_p, gr_p, v_table, u_table)

    nblk = 10
    rows = _BP // nblk
    out = pl.pallas_call(
        _tc_body,
        grid=(nblk,),
        in_specs=[
            pl.BlockSpec((rows, _D), lambda i: (i, 0)),
            pl.BlockSpec((rows, _D), lambda i: (i, 0)),
            pl.BlockSpec((rows, _D), lambda i: (i, 0)),
            pl.BlockSpec((_D, _D), lambda i: (0, 0)),
            pl.BlockSpec((_D, _D), lambda i: (0, 0)),
            pl.BlockSpec((_D, _D), lambda i: (0, 0)),
            pl.BlockSpec((1, _D), lambda i: (0, 0)),
        ],
        out_specs=pl.BlockSpec((rows, _D), lambda i: (i, 0)),
        out_shape=jax.ShapeDtypeStruct((_BP, _D), f32),
    )(self_rows, uv_sum, gr_sum, w1, w2, w3, b2)
    return out[:_B]


def kernel(nodes, uv_neigh, graph_neigh, v_table, u_table, W, b):
    pad = _BP - _B
    nodes_p = jnp.pad(nodes, (0, pad))
    uv_p = jnp.pad(uv_neigh, ((0, pad), (0, 0))).reshape(_BP * _K)
    gr_p = jnp.pad(graph_neigh, ((0, pad), (0, 0))).reshape(_BP * _K)
    w1 = 0.3 * W[:_D]
    w2 = (0.4 / _K) * W[_D:2 * _D]
    w3 = 0.3 * W[2 * _D:]
    b2 = b.reshape(1, _D)
    return _run(nodes_p, uv_p, gr_p, v_table, u_table, w1, w2, w3, b2)


# speedup vs baseline: 3.2938x; 1.0497x over previous
import jax
import jax.numpy as jnp
from jax import lax
from jax.experimental import pallas as pl
from jax.experimental.pallas import tpu as pltpu
from jax.experimental.pallas import tpu_sc as plsc

_B = 10000
_K = 64
_D = 128
_L = 16
_NC = 2
_NS = 16
_NW = _NC * _NS
_BP = 10240
_BW = _BP // _NW
_GROWS = 2
_GIDX = _GROWS * _K
_NG = _BW // _GROWS
_NBUF = 4


_ONLY_CORE = -1
_ABL_NO_ACCUM = False
_ABL_NO_GATHER = False


def _sc_body(nodes_hbm, uv_hbm, gr_hbm, vtab_hbm, utab_hbm,
             self_out, uv_out, gr_out,
             idx_v, bufs, stage_v, sems):
    if _ONLY_CORE < 0:
        wid = lax.axis_index("s") * _NC + lax.axis_index("c")
        _sc_worker(wid * _BW,
                   nodes_hbm, uv_hbm, gr_hbm, vtab_hbm, utab_hbm,
                   self_out, uv_out, gr_out,
                   idx_v, bufs, stage_v, sems)
    else:
        @pl.when(lax.axis_index("c") == _ONLY_CORE)
        def _():
            s = lax.axis_index("s")
            for h in range(2):
                _sc_worker((s * 2 + h) * _BW,
                           nodes_hbm, uv_hbm, gr_hbm, vtab_hbm, utab_hbm,
                           self_out, uv_out, gr_out,
                           idx_v, bufs, stage_v, sems)


def _sc_worker(base, nodes_hbm, uv_hbm, gr_hbm, vtab_hbm, utab_hbm,
               self_out, uv_out, gr_out,
               idx_v, bufs, stage_v, sems):

    pltpu.sync_copy(nodes_hbm.at[pl.ds(base, _BW)], idx_v.at[pl.ds(0, _BW)])
    c0 = pltpu.async_copy(
        vtab_hbm.at[idx_v.at[pl.ds(0, 128)]], stage_v.at[pl.ds(0, 128)], sems[0])
    c1 = pltpu.async_copy(
        vtab_hbm.at[idx_v.at[pl.ds(128, 128)]], stage_v.at[pl.ds(128, 128)],
        sems[1])
    c2 = pltpu.async_copy(
        vtab_hbm.at[idx_v.at[pl.ds(256, 64)]], stage_v.at[pl.ds(256, 64)],
        sems[0])
    c0.wait()
    c2.wait()
    c1.wait()
    pltpu.sync_copy(stage_v, self_out.at[pl.ds(base, _BW)])

    def seg_sum(idx_hbm, tab_hbm, out_hbm):
        pltpu.sync_copy(idx_hbm.at[pl.ds(base * _K, _BW * _K)], idx_v)

        def fire(g, p):
            if _ABL_NO_GATHER:
                pltpu.async_copy(
                    tab_hbm.at[pl.ds(g * _GIDX, _GIDX)], bufs[p], sems[p])
            else:
                pltpu.async_copy(
                    tab_hbm.at[idx_v.at[pl.ds(g * _GIDX, _GIDX)]], bufs[p],
                    sems[p])

        def drain(g, p):
            if _ABL_NO_GATHER:
                pltpu.make_async_copy(
                    tab_hbm.at[pl.ds(g * _GIDX, _GIDX)], bufs[p],
                    sems[p]).wait()
            else:
                pltpu.make_async_copy(
                    tab_hbm.at[idx_v.at[pl.ds(g * _GIDX, _GIDX)]], bufs[p],
                    sems[p]).wait()

        def accum_store(g, p):
            if _ABL_NO_ACCUM:
                return
            buf = bufs[p]
            for rr in range(_GROWS):
                def body(j, accs, rr=rr):
                    r0 = rr * _K + j * 8
                    accs = list(accs)
                    for u in range(8):
                        for c in range(_D // _L):
                            accs[c] = accs[c] + buf[r0 + u, pl.ds(c * _L, _L)]
                    return tuple(accs)
                accs = lax.fori_loop(
                    0, _K // 8, body,
                    tuple(jnp.zeros((_L,), jnp.float32)
                          for _ in range(_D // _L)))
                for c in range(_D // _L):
                    stage_v[g * _GROWS + rr, pl.ds(c * _L, _L)] = accs[c]

        for p in range(_NBUF):
            fire(p, p)

        @pl.loop(0, _NG - _NBUF, step=_NBUF)
        def _(g):
            for p in range(_NBUF):
                gg = g + p
                drain(gg, p)
                accum_store(gg, p)
                fire(gg + _NBUF, p)

        for p in range(_NBUF):
            gg = _NG - _NBUF + p
            drain(gg, p)
            accum_store(gg, p)

        pltpu.sync_copy(stage_v, out_hbm.at[pl.ds(base, _BW)])

    seg_sum(uv_hbm, utab_hbm, uv_out)
    seg_sum(gr_hbm, vtab_hbm, gr_out)


def _tc_body(self_ref, us_ref, gs_ref, w1_ref, w2_ref, w3_ref, b_ref, o_ref):
    s = self_ref[...]
    u = us_ref[...]
    g = gs_ref[...] * (1.0 / _K)
    g = jnp.where(g >= 0.0, jnp.floor(g), jnp.ceil(g))
    acc = jnp.dot(s, w1_ref[...], preferred_element_type=jnp.float32)
    acc = acc + jnp.dot(u, w2_ref[...], preferred_element_type=jnp.float32)
    acc = acc + jnp.dot(g, w3_ref[...], preferred_element_type=jnp.float32)
    o_ref[...] = jnp.maximum(acc + b_ref[...], 0.0)


@jax.jit
def _run(nodes_p, uv_p, gr_p, v_table, u_table, w1, w2, w3, b2):
    mesh = plsc.VectorSubcoreMesh(core_axis_name="c", subcore_axis_name="s")
    f32 = jnp.float32
    sc = pl.kernel(
        _sc_body,
        out_type=[
            jax.ShapeDtypeStruct((_BP, _D), f32),
            jax.ShapeDtypeStruct((_BP, _D), f32),
            jax.ShapeDtypeStruct((_BP, _D), f32),
        ],
        mesh=mesh,
        scratch_types=[
            pltpu.VMEM((_BW * _K,), jnp.int32),
            tuple(pltpu.VMEM((_GIDX, _D), f32) for _ in range(_NBUF)),
            pltpu.VMEM((_BW, _D), f32),
            tuple(pltpu.SemaphoreType.DMA for _ in range(_NBUF)),
        ],
    )
    self_rows, uv_sum, gr_sum = sc(nodes_p, uv_p, gr_p, v_table, u_table)

    nblk = 10
    rows = _BP // nblk
    out = pl.pallas_call(
        _tc_body,
        grid=(nblk,),
        in_specs=[
            pl.BlockSpec((rows, _D), lambda i: (i, 0)),
            pl.BlockSpec((rows, _D), lambda i: (i, 0)),
            pl.BlockSpec((rows, _D), lambda i: (i, 0)),
            pl.BlockSpec((_D, _D), lambda i: (0, 0)),
            pl.BlockSpec((_D, _D), lambda i: (0, 0)),
            pl.BlockSpec((_D, _D), lambda i: (0, 0)),
            pl.BlockSpec((1, _D), lambda i: (0, 0)),
        ],
        out_specs=pl.BlockSpec((rows, _D), lambda i: (i, 0)),
        out_shape=jax.ShapeDtypeStruct((_BP, _D), f32),
    )(self_rows, uv_sum, gr_sum, w1, w2, w3, b2)
    return out[:_B]


def kernel(nodes, uv_neigh, graph_neigh, v_table, u_table, W, b):
    pad = _BP - _B
    nodes_p = jnp.pad(nodes, (0, pad))
    uv_p = jnp.pad(uv_neigh, ((0, pad), (0, 0))).reshape(_BP * _K)
    gr_p = jnp.pad(graph_neigh, ((0, pad), (0, 0))).reshape(_BP * _K)
    w1 = 0.3 * W[:_D]
    w2 = (0.4 / _K) * W[_D:2 * _D]
    w3 = 0.3 * W[2 * _D:]
    b2 = b.reshape(1, _D)
    return _run(nodes_p, uv_p, gr_p, v_table, u_table, w1, w2, w3, b2)
